# baseline (speedup 1.0000x reference)
import functools
import jax
import jax.numpy as jnp
from jax import lax
from jax.experimental import pallas as pl
from jax.experimental.pallas import tpu as pltpu
from jax.experimental.pallas import tpu_sc as plsc

_N2 = 20000
_NT = 26989
_R = 20
_NB = 5
_D = 200
_B = 1024
_S = 100

_SLAB = 8000
_CH = 1024
_GR = 64
_NTILE = 16


def _make_segsum(e_pad, nslab, ns_per_sc, v_rows):
    e_tile = e_pad // _NTILE
    nch = e_tile // _CH
    assert e_tile % _CH == 0
    kpad = nslab * _SLAB
    mesh = plsc.VectorSubcoreMesh(core_axis_name="c", subcore_axis_name="s")

    @functools.partial(
        pl.kernel,
        out_type=(
            jax.ShapeDtypeStruct((kpad, _D), jnp.float32),
            jax.ShapeDtypeStruct((kpad, 8), jnp.float32),
        ),
        mesh=mesh,
        compiler_params=pltpu.CompilerParams(use_tc_tiling_on_sc=False, needs_layout_passes=False),
        scratch_types=[
            pltpu.VMEM_SHARED((_SLAB + 8, _D), jnp.float32),
            pltpu.VMEM_SHARED((_SLAB + 8, 8), jnp.float32),
            pltpu.VMEM((_CH,), jnp.int32),
            pltpu.VMEM((_CH,), jnp.int32),
            pltpu.VMEM((_CH + _GR,), jnp.int32),
            pltpu.VMEM((_CH + _GR,), jnp.int32),
            pltpu.VMEM((1, _GR), jnp.int32),
            pltpu.VMEM((_GR, _D), jnp.float32),
            pltpu.VMEM((_GR, 8), jnp.float32),
            pltpu.VMEM((_GR, 8), jnp.float32),
            pltpu.SemaphoreType.DMA,
        ],
    )
    def seg(x_hbm, src_hbm, key_hbm, ones_hbm, zrows_hbm, zeros8_hbm,
            outsum, outcnt, acc, cacc, key_c, src_c, msrc, mlk, mlk2d,
            rows, ones_v, zeros8_v, sem):
        sc = lax.axis_index("c")
        tid = lax.axis_index("s")
        pltpu.sync_copy(ones_hbm, ones_v)
        pltpu.sync_copy(zeros8_hbm, zeros8_v)

        _zchunks = tuple((64 * i, 64) for i in range(7)) + ((448, 48),)

        def zero_stripe(dst, src64):
            for z, nr in _zchunks:
                pltpu.sync_copy(src64.at[pl.ds(0, nr)],
                                dst.at[pl.ds(tid * 496 + z, nr)])

            @pl.when(tid == 0)
            def _():
                pltpu.sync_copy(src64.at[pl.ds(0, 64)],
                                dst.at[pl.ds(7936, 64)])

        def copyout_stripe(src, dst, lo):
            for z, nr in _zchunks:
                pltpu.sync_copy(src.at[pl.ds(tid * 496 + z, nr)],
                                dst.at[pl.ds(lo + tid * 496 + z, nr)])

            @pl.when(tid == 0)
            def _():
                pltpu.sync_copy(src.at[pl.ds(7936, 64)],
                                dst.at[pl.ds(lo + 7936, 64)])

        def slab_body(si, carry):
            slab = sc * ns_per_sc + si
            lo = slab * _SLAB
            pltpu.sync_copy(zrows_hbm, rows)
            zero_stripe(acc, rows)
            zero_stripe(cacc, zeros8_v)
            plsc.subcore_barrier()

            def chunk_body(ci, carry2):
                ebase = tid * e_tile + ci * _CH
                pltpu.sync_copy(key_hbm.at[pl.ds(ebase, _CH)], key_c)
                pltpu.sync_copy(src_hbm.at[pl.ds(ebase, _CH)], src_c)

                def filt_body(v, off):
                    kv = key_c[pl.ds(v * 16, 16)]
                    sv = src_c[pl.ds(v * 16, 16)]
                    lk = kv - lo
                    m = (lk >= 0) & (lk < _SLAB)
                    idx = off + plsc.cumsum(m.astype(jnp.int32)) - 1
                    plsc.store_scatter(msrc, [idx], sv, mask=m)
                    plsc.store_scatter(mlk, [idx], lk, mask=m)
                    npop = plsc.all_reduce_population_count(m)
                    return off + npop[0]

                off_c = lax.fori_loop(0, _CH // 16, filt_body,
                                      jnp.int32(0))
                io16 = lax.iota(jnp.int32, 16)
                for t in range(_GR // 16):
                    idxf = off_c + t * 16 + io16
                    plsc.store_scatter(msrc, [idxf],
                                       jnp.zeros((16,), jnp.int32))
                    plsc.store_scatter(mlk, [idxf],
                                       jnp.full((16,), _SLAB, jnp.int32))
                ngroups = (off_c + _GR - 1) // _GR
                for g in range(_CH // _GR + 1):
                    @pl.when(g < ngroups)
                    def _():
                        for k in range(_GR // 16):
                            mlk2d[0, pl.ds(k * 16, 16)] = (
                                mlk[pl.ds(g * _GR + k * 16, 16)])
                        pltpu.async_copy(
                            x_hbm.at[msrc.at[pl.ds(g * _GR, _GR)]],
                            rows, sem).wait()
                        pltpu.sync_copy(rows, acc.at[mlk2d.at[0]],
                                        add=True)
                        pltpu.sync_copy(ones_v, cacc.at[mlk2d.at[0]],
                                        add=True)
                return carry2

            lax.fori_loop(0, nch, chunk_body, jnp.int32(0))
            plsc.subcore_barrier()
            copyout_stripe(acc, outsum, lo)
            copyout_stripe(cacc, outcnt, lo)
            plsc.subcore_barrier()
            return carry

        lax.fori_loop(0, ns_per_sc, slab_body, jnp.int32(0))

    return seg


def _make_gather(nrows, gr):
    share = nrows // 32
    ngr = share // gr
    assert share % gr == 0
    mesh = plsc.VectorSubcoreMesh(core_axis_name="c", subcore_axis_name="s")

    def body(tab_hbm, idx_hbm, out_hbm, idx_v, rows, sem):
        w = lax.axis_index("s") * 2 + lax.axis_index("c")
        pltpu.sync_copy(idx_hbm.at[pl.ds(w * share, share)], idx_v)
        for g in range(ngr):
            pltpu.async_copy(tab_hbm.at[idx_v.at[pl.ds(g * gr, gr)]],
                             rows, sem).wait()
            pltpu.sync_copy(rows,
                            out_hbm.at[pl.ds(w * share + g * gr, gr)])

    def make(dty):
        return pl.kernel(
            body,
            out_type=jax.ShapeDtypeStruct((nrows, _D), dty),
            mesh=mesh,
            compiler_params=pltpu.CompilerParams(use_tc_tiling_on_sc=False, needs_layout_passes=False),
            scratch_types=[
                pltpu.VMEM((share,), jnp.int32),
                pltpu.VMEM((gr, _D), dty),
                pltpu.SemaphoreType.DMA,
            ],
        )
    return make(jnp.float32)


def _xg1_body(a_ref, s_ref, c_ref, o_ref):
    cnt = jnp.maximum(c_ref[:, :1], 1.0)
    o_ref[...] = a_ref[...] + s_ref[...] / cnt


def _rgcn_body(s_ref, c_ref, x_ref, coeff_ref, bases_ref, root_ref,
               bias_ref, o_ref, *, relu):
    mean = s_ref[...] / jnp.maximum(c_ref[...][..., :1], 1.0)
    out = jnp.dot(x_ref[...], root_ref[...],
                  preferred_element_type=jnp.float32)
    coeff = coeff_ref[...]
    for b in range(_NB):
        cb = coeff[:, b].reshape(_R, 1, 1)
        aggb = jnp.sum(mean * cb, axis=0)
        out = out + jnp.dot(aggb, bases_ref[b],
                            preferred_element_type=jnp.float32)
    out = out + bias_ref[...]
    if relu:
        out = jnp.maximum(out, 0.0)
    o_ref[...] = out


def _pred_body(xm_ref, se_ref, w_ref, o_ref):
    xm = xm_ref[...]
    se = jnp.clip(se_ref[...], 0.0, 1.0)
    w = jnp.clip(w_ref[...], 0.0, 1.0)
    xsq = (xm * xm) * w[:, 0][None, :]
    o_ref[...] = jax.nn.sigmoid(
        jnp.einsum('bsd,bd->bs', se, xsq,
                   preferred_element_type=jnp.float32))


def _pad_edges(src, key, e_pad, sentinel):
    e = src.shape[0]
    src_p = jnp.concatenate(
        [src, jnp.zeros((e_pad - e,), jnp.int32)])
    key_p = jnp.concatenate(
        [key, jnp.full((e_pad - e,), sentinel, jnp.int32)])
    return src_p, key_p


_E1_PAD = 25 * _NTILE * _CH
_E2_PAD = 20 * _NTILE * _CH
_NSLAB1 = 4
_NSLAB2 = 50
_KP1 = _NSLAB1 * _SLAB
_KP2 = _NSLAB2 * _SLAB

_seg1 = _make_segsum(_E1_PAD, _NSLAB1, 2, _NT)
_seg2a = _make_segsum(_E2_PAD, _NSLAB2, 25, _KP1)
_seg2b = _make_segsum(_E2_PAD, _NSLAB2, 25, _N2)
_gat_se = _make_gather(_B * _S, _GR)
_gat_xm = _make_gather(_B, _B // 32)


def kernel(all_node_embedding, bases1, coeff1, root1, bias1, bases2, coeff2,
           root2, bias2, weights, edge_index_g2, edge_type_g2, edge_index_g1,
           index_list, sample_index, sample_index_min):
    f32 = jnp.float32
    aemb = all_node_embedding
    ones_in = jnp.ones((_GR, 8), f32)
    zrows_in = jnp.zeros((_GR, _D), f32)
    zeros8_in = jnp.zeros((_GR, 8), f32)

    src1 = edge_index_g1[0].astype(jnp.int32)
    key1 = edge_index_g1[1].astype(jnp.int32)
    src1p, key1p = _pad_edges(src1, key1, _E1_PAD, _KP1)
    sum1, cnt1 = _seg1(aemb, src1p, key1p, ones_in, zrows_in, zeros8_in)

    aemb_p = jnp.concatenate(
        [aemb, jnp.zeros((_KP1 - _NT, _D), f32)])
    x_g1p = pl.pallas_call(
        _xg1_body,
        grid=(_KP1 // 800,),
        in_specs=[
            pl.BlockSpec((800, _D), lambda i: (i, 0)),
            pl.BlockSpec((800, _D), lambda i: (i, 0)),
            pl.BlockSpec((800, 8), lambda i: (i, 0)),
        ],
        out_specs=pl.BlockSpec((800, _D), lambda i: (i, 0)),
        out_shape=jax.ShapeDtypeStruct((_KP1, _D), f32),
    )(aemb_p, sum1, cnt1)

    src2 = edge_index_g2[0].astype(jnp.int32)
    key2 = (edge_type_g2.astype(jnp.int32) * _N2
            + edge_index_g2[1].astype(jnp.int32))
    src2p, key2p = _pad_edges(src2, key2, _E2_PAD, _KP2)

    bn = 400
    def rgcn_dense(sums, cnts, x_in, coeff, bases, root, bias, relu):
        body = functools.partial(_rgcn_body, relu=relu)
        return pl.pallas_call(
            body,
            grid=(_N2 // bn,),
            in_specs=[
                pl.BlockSpec((_R, bn, _D), lambda i: (0, i, 0)),
                pl.BlockSpec((_R, bn, 8), lambda i: (0, i, 0)),
                pl.BlockSpec((bn, _D), lambda i: (i, 0)),
                pl.BlockSpec((_R, _NB), lambda i: (0, 0)),
                pl.BlockSpec((_NB, _D, _D), lambda i: (0, 0, 0)),
                pl.BlockSpec((_D, _D), lambda i: (0, 0)),
                pl.BlockSpec((1, _D), lambda i: (0, 0)),
            ],
            out_specs=pl.BlockSpec((bn, _D), lambda i: (i, 0)),
            out_shape=jax.ShapeDtypeStruct((_N2, _D), f32),
        )(sums.reshape(_R, _N2, _D), cnts.reshape(_R, _N2, 8), x_in,
          coeff, bases, root, bias.reshape(1, _D))

    sum2, cnt2 = _seg2a(x_g1p, src2p, key2p, ones_in, zrows_in, zeros8_in)
    h = rgcn_dense(sum2, cnt2, x_g1p[:_N2], coeff1, bases1, root1, bias1,
                   True)
    sum3, cnt3 = _seg2b(h, src2p, key2p, ones_in, zrows_in, zeros8_in)
    h = rgcn_dense(sum3, cnt3, h, coeff2, bases2, root2, bias2, False)

    se_idx = (sample_index_min.astype(jnp.int32) + _N2).reshape(-1)
    se_rows = _gat_se(x_g1p, se_idx)
    x_mini = _gat_xm(h, index_list.astype(jnp.int32))

    bb = 128
    out = pl.pallas_call(
        _pred_body,
        grid=(_B // bb,),
        in_specs=[
            pl.BlockSpec((bb, _D), lambda i: (i, 0)),
            pl.BlockSpec((bb, _S, _D), lambda i: (i, 0, 0)),
            pl.BlockSpec((_D, 1), lambda i: (0, 0)),
        ],
        out_specs=pl.BlockSpec((bb, _S), lambda i: (i, 0)),
        out_shape=jax.ShapeDtypeStruct((_B, _S), f32),
    )(x_mini, se_rows.reshape(_B, _S, _D), weights)
    return out


# speedup vs baseline: 3.0416x; 3.0416x over previous
import functools
import jax
import jax.numpy as jnp
from jax import lax
from jax.experimental import pallas as pl
from jax.experimental.pallas import tpu as pltpu
from jax.experimental.pallas import tpu_sc as plsc

_N2 = 20000
_NT = 26989
_R = 20
_NB = 5
_D = 200
_B = 1024
_S = 100

_SLAB = 8000
_CH = 2048
_GR = 64
_NTILE = 16


def _make_segsum(e_pad, nslab, ns_per_sc, v_rows, with_cnt=True):
    e_tile = e_pad // _NTILE
    nch = e_tile // _CH
    assert e_tile % _CH == 0
    kpad = nslab * _SLAB
    mesh = plsc.VectorSubcoreMesh(core_axis_name="c", subcore_axis_name="s")

    @functools.partial(
        pl.kernel,
        out_type=(
            jax.ShapeDtypeStruct((kpad, _D), jnp.float32),
            jax.ShapeDtypeStruct((kpad, 8), jnp.float32),
        ),
        mesh=mesh,
        compiler_params=pltpu.CompilerParams(use_tc_tiling_on_sc=False, needs_layout_passes=False),
        scratch_types=[
            pltpu.VMEM_SHARED((_SLAB + 8, _D), jnp.float32),
            pltpu.VMEM_SHARED((_SLAB + 8, 8), jnp.float32),
            pltpu.VMEM((_CH,), jnp.int32),
            pltpu.VMEM((_CH,), jnp.int32),
            pltpu.VMEM((_CH + _GR,), jnp.int32),
            pltpu.VMEM((_CH + _GR,), jnp.int32),
            pltpu.VMEM((1, _GR), jnp.int32),
            pltpu.VMEM((_GR, _D), jnp.float32),
            pltpu.VMEM((_GR, 8), jnp.float32),
            pltpu.VMEM((_GR, 8), jnp.float32),
            pltpu.SemaphoreType.DMA,
        ],
    )
    def seg(x_hbm, src_hbm, key_hbm, ones_hbm, zrows_hbm, zeros8_hbm,
            outsum, outcnt, acc, cacc, key_c, src_c, msrc, mlk, mlk2d,
            rows, ones_v, zeros8_v, sem):
        sc = lax.axis_index("c")
        tid = lax.axis_index("s")
        pltpu.sync_copy(ones_hbm, ones_v)
        pltpu.sync_copy(zeros8_hbm, zeros8_v)

        _zchunks = tuple((64 * i, 64) for i in range(7)) + ((448, 48),)

        def zero_stripe(dst, src64):
            for z, nr in _zchunks:
                pltpu.sync_copy(src64.at[pl.ds(0, nr)],
                                dst.at[pl.ds(tid * 496 + z, nr)])

            @pl.when(tid == 0)
            def _():
                pltpu.sync_copy(src64.at[pl.ds(0, 64)],
                                dst.at[pl.ds(7936, 64)])

        def copyout_stripe(src, dst, lo):
            for z, nr in _zchunks:
                pltpu.sync_copy(src.at[pl.ds(tid * 496 + z, nr)],
                                dst.at[pl.ds(lo + tid * 496 + z, nr)])

            @pl.when(tid == 0)
            def _():
                pltpu.sync_copy(src.at[pl.ds(7936, 64)],
                                dst.at[pl.ds(lo + 7936, 64)])

        def slab_body(si, carry):
            slab = sc * ns_per_sc + si
            lo = slab * _SLAB
            pltpu.sync_copy(zrows_hbm, rows)
            zero_stripe(acc, rows)
            if with_cnt:
                zero_stripe(cacc, zeros8_v)
            plsc.subcore_barrier()

            def chunk_body(ci, carry2):
                ebase = tid * e_tile + ci * _CH
                pltpu.sync_copy(key_hbm.at[pl.ds(ebase, _CH)], key_c)
                pltpu.sync_copy(src_hbm.at[pl.ds(ebase, _CH)], src_c)

                def filt_body(v, off):
                    kv0 = key_c[pl.ds(v * 32, 16)]
                    sv0 = src_c[pl.ds(v * 32, 16)]
                    kv1 = key_c[pl.ds(v * 32 + 16, 16)]
                    sv1 = src_c[pl.ds(v * 32 + 16, 16)]
                    lk0 = kv0 - lo
                    lk1 = kv1 - lo
                    m0 = (lk0 >= 0) & (lk0 < _SLAB)
                    m1 = (lk1 >= 0) & (lk1 < _SLAB)
                    cs0 = plsc.cumsum(m0.astype(jnp.int32))
                    cs1 = plsc.cumsum(m1.astype(jnp.int32))
                    t0 = cs0[15]
                    plsc.store_scatter(msrc, [off + cs0 - 1], sv0, mask=m0)
                    plsc.store_scatter(mlk, [off + cs0 - 1], lk0, mask=m0)
                    off1 = off + t0
                    plsc.store_scatter(msrc, [off1 + cs1 - 1], sv1, mask=m1)
                    plsc.store_scatter(mlk, [off1 + cs1 - 1], lk1, mask=m1)
                    return off1 + cs1[15]

                off_c = lax.fori_loop(0, _CH // 32, filt_body,
                                      jnp.int32(0))
                io16 = lax.iota(jnp.int32, 16)
                for t in range(_GR // 16):
                    idxf = off_c + t * 16 + io16
                    plsc.store_scatter(msrc, [idxf],
                                       jnp.zeros((16,), jnp.int32))
                    plsc.store_scatter(mlk, [idxf],
                                       jnp.full((16,), _SLAB, jnp.int32))
                ngroups = (off_c + _GR - 1) // _GR
                for g in range(_CH // _GR + 1):
                    @pl.when(g < ngroups)
                    def _():
                        for k in range(_GR // 16):
                            mlk2d[0, pl.ds(k * 16, 16)] = (
                                mlk[pl.ds(g * _GR + k * 16, 16)])
                        pltpu.async_copy(
                            x_hbm.at[msrc.at[pl.ds(g * _GR, _GR)]],
                            rows, sem).wait()
                        pltpu.sync_copy(rows, acc.at[mlk2d.at[0]],
                                        add=True)
                        pltpu.sync_copy(ones_v, cacc.at[mlk2d.at[0]],
                                        add=True)
                return carry2

            lax.fori_loop(0, nch, chunk_body, jnp.int32(0))
            plsc.subcore_barrier()
            copyout_stripe(acc, outsum, lo)
            if with_cnt:
                copyout_stripe(cacc, outcnt, lo)
            plsc.subcore_barrier()
            return carry

        lax.fori_loop(0, ns_per_sc, slab_body, jnp.int32(0))

    return seg


def _make_gather(nrows, gr):
    share = nrows // 32
    ngr = share // gr
    assert share % gr == 0
    mesh = plsc.VectorSubcoreMesh(core_axis_name="c", subcore_axis_name="s")

    def body(tab_hbm, idx_hbm, out_hbm, idx_v, rows, sem):
        w = lax.axis_index("s") * 2 + lax.axis_index("c")
        pltpu.sync_copy(idx_hbm.at[pl.ds(w * share, share)], idx_v)
        for g in range(ngr):
            pltpu.async_copy(tab_hbm.at[idx_v.at[pl.ds(g * gr, gr)]],
                             rows, sem).wait()
            pltpu.sync_copy(rows,
                            out_hbm.at[pl.ds(w * share + g * gr, gr)])

    def make(dty):
        return pl.kernel(
            body,
            out_type=jax.ShapeDtypeStruct((nrows, _D), dty),
            mesh=mesh,
            compiler_params=pltpu.CompilerParams(use_tc_tiling_on_sc=False, needs_layout_passes=False),
            scratch_types=[
                pltpu.VMEM((share,), jnp.int32),
                pltpu.VMEM((gr, _D), dty),
                pltpu.SemaphoreType.DMA,
            ],
        )
    return make(jnp.float32)


def _xg1_body(a_ref, s_ref, c_ref, o_ref):
    cnt = jnp.maximum(c_ref[:, :1], 1.0)
    o_ref[...] = a_ref[...] + s_ref[...] / cnt


def _rgcn_body(s_ref, c_ref, x_ref, coeff_ref, bases_ref, root_ref,
               bias_ref, o_ref, *, relu):
    mean = s_ref[...] / jnp.maximum(c_ref[...][..., :1], 1.0)
    out = jnp.dot(x_ref[...], root_ref[...],
                  preferred_element_type=jnp.float32)
    coeff = coeff_ref[...]
    for b in range(_NB):
        cb = coeff[:, b].reshape(_R, 1, 1)
        aggb = jnp.sum(mean * cb, axis=0)
        out = out + jnp.dot(aggb, bases_ref[b],
                            preferred_element_type=jnp.float32)
    out = out + bias_ref[...]
    if relu:
        out = jnp.maximum(out, 0.0)
    o_ref[...] = out


def _pred_body(xm_ref, se_ref, w_ref, o_ref):
    xm = xm_ref[...]
    se = jnp.clip(se_ref[...], 0.0, 1.0)
    w = jnp.clip(w_ref[...], 0.0, 1.0)
    xsq = (xm * xm) * w[:, 0][None, :]
    o_ref[...] = jax.nn.sigmoid(
        jnp.einsum('bsd,bd->bs', se, xsq,
                   preferred_element_type=jnp.float32))


def _pad_edges(src, key, e_pad, sentinel):
    e = src.shape[0]
    src_p = jnp.concatenate(
        [src, jnp.zeros((e_pad - e,), jnp.int32)])
    key_p = jnp.concatenate(
        [key, jnp.full((e_pad - e,), sentinel, jnp.int32)])
    return src_p, key_p


_E1_PAD = 13 * _NTILE * _CH
_E2_PAD = 10 * _NTILE * _CH
_NSLAB1 = 4
_NSLAB2 = 50
_KP1 = _NSLAB1 * _SLAB
_KP2 = _NSLAB2 * _SLAB

_seg1 = _make_segsum(_E1_PAD, _NSLAB1, 2, _NT)
_seg2a = _make_segsum(_E2_PAD, _NSLAB2, 25, _KP1)
_seg2b = _make_segsum(_E2_PAD, _NSLAB2, 25, _N2, with_cnt=False)
_gat_se = _make_gather(_B * _S, _GR)
_gat_xm = _make_gather(_B, _B // 32)


def kernel(all_node_embedding, bases1, coeff1, root1, bias1, bases2, coeff2,
           root2, bias2, weights, edge_index_g2, edge_type_g2, edge_index_g1,
           index_list, sample_index, sample_index_min):
    f32 = jnp.float32
    aemb = all_node_embedding
    ones_in = jnp.ones((_GR, 8), f32)
    zrows_in = jnp.zeros((_GR, _D), f32)
    zeros8_in = jnp.zeros((_GR, 8), f32)

    src1 = edge_index_g1[0].astype(jnp.int32)
    key1 = edge_index_g1[1].astype(jnp.int32)
    src1p, key1p = _pad_edges(src1, key1, _E1_PAD, _KP1)
    sum1, cnt1 = _seg1(aemb, src1p, key1p, ones_in, zrows_in, zeros8_in)

    aemb_p = jnp.concatenate(
        [aemb, jnp.zeros((_KP1 - _NT, _D), f32)])
    x_g1p = pl.pallas_call(
        _xg1_body,
        grid=(_KP1 // 800,),
        in_specs=[
            pl.BlockSpec((800, _D), lambda i: (i, 0)),
            pl.BlockSpec((800, _D), lambda i: (i, 0)),
            pl.BlockSpec((800, 8), lambda i: (i, 0)),
        ],
        out_specs=pl.BlockSpec((800, _D), lambda i: (i, 0)),
        out_shape=jax.ShapeDtypeStruct((_KP1, _D), f32),
    )(aemb_p, sum1, cnt1)

    src2 = edge_index_g2[0].astype(jnp.int32)
    key2 = (edge_type_g2.astype(jnp.int32) * _N2
            + edge_index_g2[1].astype(jnp.int32))
    src2p, key2p = _pad_edges(src2, key2, _E2_PAD, _KP2)

    bn = 400
    def rgcn_dense(sums, cnts, x_in, coeff, bases, root, bias, relu):
        body = functools.partial(_rgcn_body, relu=relu)
        return pl.pallas_call(
            body,
            grid=(_N2 // bn,),
            in_specs=[
                pl.BlockSpec((_R, bn, _D), lambda i: (0, i, 0)),
                pl.BlockSpec((_R, bn, 8), lambda i: (0, i, 0)),
                pl.BlockSpec((bn, _D), lambda i: (i, 0)),
                pl.BlockSpec((_R, _NB), lambda i: (0, 0)),
                pl.BlockSpec((_NB, _D, _D), lambda i: (0, 0, 0)),
                pl.BlockSpec((_D, _D), lambda i: (0, 0)),
                pl.BlockSpec((1, _D), lambda i: (0, 0)),
            ],
            out_specs=pl.BlockSpec((bn, _D), lambda i: (i, 0)),
            out_shape=jax.ShapeDtypeStruct((_N2, _D), f32),
        )(sums.reshape(_R, _N2, _D), cnts.reshape(_R, _N2, 8), x_in,
          coeff, bases, root, bias.reshape(1, _D))

    sum2, cnt2 = _seg2a(x_g1p, src2p, key2p, ones_in, zrows_in, zeros8_in)
    h = rgcn_dense(sum2, cnt2, x_g1p[:_N2], coeff1, bases1, root1, bias1,
                   True)
    sum3, _ = _seg2b(h, src2p, key2p, ones_in, zrows_in, zeros8_in)
    h = rgcn_dense(sum3, cnt2, h, coeff2, bases2, root2, bias2, False)

    se_idx = (sample_index_min.astype(jnp.int32) + _N2).reshape(-1)
    se_rows = _gat_se(x_g1p, se_idx)
    x_mini = _gat_xm(h, index_list.astype(jnp.int32))

    bb = 128
    out = pl.pallas_call(
        _pred_body,
        grid=(_B // bb,),
        in_specs=[
            pl.BlockSpec((bb, _D), lambda i: (i, 0)),
            pl.BlockSpec((bb, _S, _D), lambda i: (i, 0, 0)),
            pl.BlockSpec((_D, 1), lambda i: (0, 0)),
        ],
        out_specs=pl.BlockSpec((bb, _S), lambda i: (i, 0)),
        out_shape=jax.ShapeDtypeStruct((_B, _S), f32),
    )(x_mini, se_rows.reshape(_B, _S, _D), weights)
    return out


# speedup vs baseline: 3.0539x; 1.0041x over previous
import functools
import jax
import jax.numpy as jnp
from jax import lax
from jax.experimental import pallas as pl
from jax.experimental.pallas import tpu as pltpu
from jax.experimental.pallas import tpu_sc as plsc

_N2 = 20000
_NT = 26989
_R = 20
_NB = 5
_D = 200
_B = 1024
_S = 100

_SLAB = 8000
_CH = 2048
_GR = 64
_NTILE = 16


def _make_segsum(e_pad, nslab, ns_per_sc, v_rows, with_cnt=True):
    e_tile = e_pad // _NTILE
    nch = e_tile // _CH
    assert e_tile % _CH == 0
    kpad = nslab * _SLAB
    mesh = plsc.VectorSubcoreMesh(core_axis_name="c", subcore_axis_name="s")

    @functools.partial(
        pl.kernel,
        out_type=(
            jax.ShapeDtypeStruct((kpad, _D), jnp.float32),
            jax.ShapeDtypeStruct((kpad, 8), jnp.float32),
        ),
        mesh=mesh,
        compiler_params=pltpu.CompilerParams(use_tc_tiling_on_sc=False, needs_layout_passes=False),
        scratch_types=[
            pltpu.VMEM_SHARED((_SLAB + 8, _D), jnp.float32),
            pltpu.VMEM_SHARED((_SLAB + 8, 8), jnp.float32),
            pltpu.VMEM((2 * _CH,), jnp.int32),
            pltpu.VMEM((_CH + _GR,), jnp.int32),
            pltpu.VMEM((_CH + _GR,), jnp.int32),
            pltpu.VMEM((1, _GR), jnp.int32),
            pltpu.VMEM((_GR, _D), jnp.float32),
            pltpu.VMEM((_GR, 8), jnp.float32),
            pltpu.VMEM((_GR, 8), jnp.float32),
            pltpu.SemaphoreType.DMA,
        ],
    )
    def seg(x_hbm, edges_hbm, ones_hbm, zrows_hbm, zeros8_hbm,
            outsum, outcnt, acc, cacc, ec, msrc, mlk, mlk2d,
            rows, ones_v, zeros8_v, sem):
        sc = lax.axis_index("c")
        tid = lax.axis_index("s")
        pltpu.sync_copy(ones_hbm, ones_v)
        pltpu.sync_copy(zeros8_hbm, zeros8_v)

        _zchunks = tuple((64 * i, 64) for i in range(7)) + ((448, 48),)

        def zero_stripe(dst, src64):
            for z, nr in _zchunks:
                pltpu.sync_copy(src64.at[pl.ds(0, nr)],
                                dst.at[pl.ds(tid * 496 + z, nr)])

            @pl.when(tid == 0)
            def _():
                pltpu.sync_copy(src64.at[pl.ds(0, 64)],
                                dst.at[pl.ds(7936, 64)])

        def copyout_stripe(src, dst, lo):
            for z, nr in _zchunks:
                pltpu.sync_copy(src.at[pl.ds(tid * 496 + z, nr)],
                                dst.at[pl.ds(lo + tid * 496 + z, nr)])

            @pl.when(tid == 0)
            def _():
                pltpu.sync_copy(src.at[pl.ds(7936, 64)],
                                dst.at[pl.ds(lo + 7936, 64)])

        def slab_body(si, carry):
            slab = sc * ns_per_sc + si
            lo = slab * _SLAB
            pltpu.sync_copy(zrows_hbm, rows)
            zero_stripe(acc, rows)
            if with_cnt:
                zero_stripe(cacc, zeros8_v)
            plsc.subcore_barrier()

            def chunk_body(ci, carry2):
                ebase = 2 * (tid * e_tile + ci * _CH)
                pltpu.sync_copy(edges_hbm.at[pl.ds(ebase, 2 * _CH)], ec)

                def filt_body(v, off):
                    svs, lks, ms, css = [], [], [], []
                    for u in range(4):
                        sv = ec[pl.ds(v * 64 + u * 16, 16)]
                        kv = ec[pl.ds(_CH + v * 64 + u * 16, 16)]
                        lk = kv - lo
                        m = (lk >= 0) & (lk < _SLAB)
                        svs.append(sv)
                        lks.append(lk)
                        ms.append(m)
                        css.append(plsc.cumsum(m.astype(jnp.int32)))
                    for u in range(4):
                        idx = off + css[u] - 1
                        plsc.store_scatter(msrc, [idx], svs[u], mask=ms[u])
                        plsc.store_scatter(mlk, [idx], lks[u], mask=ms[u])
                        off = off + css[u][15]
                    return off

                off_c = lax.fori_loop(0, _CH // 64, filt_body,
                                      jnp.int32(0))
                io16 = lax.iota(jnp.int32, 16)
                for t in range(_GR // 16):
                    idxf = off_c + t * 16 + io16
                    plsc.store_scatter(msrc, [idxf],
                                       jnp.zeros((16,), jnp.int32))
                    plsc.store_scatter(mlk, [idxf],
                                       jnp.full((16,), _SLAB, jnp.int32))
                ngroups = (off_c + _GR - 1) // _GR
                for g in range(_CH // _GR + 1):
                    @pl.when(g < ngroups)
                    def _():
                        for k in range(_GR // 16):
                            mlk2d[0, pl.ds(k * 16, 16)] = (
                                mlk[pl.ds(g * _GR + k * 16, 16)])
                        pltpu.async_copy(
                            x_hbm.at[msrc.at[pl.ds(g * _GR, _GR)]],
                            rows, sem).wait()
                        pltpu.sync_copy(rows, acc.at[mlk2d.at[0]],
                                        add=True)
                        pltpu.sync_copy(ones_v, cacc.at[mlk2d.at[0]],
                                        add=True)
                return carry2

            lax.fori_loop(0, nch, chunk_body, jnp.int32(0))
            plsc.subcore_barrier()
            copyout_stripe(acc, outsum, lo)
            if with_cnt:
                copyout_stripe(cacc, outcnt, lo)
            plsc.subcore_barrier()
            return carry

        lax.fori_loop(0, ns_per_sc, slab_body, jnp.int32(0))

    return seg


def _make_gather(nrows, gr):
    share = nrows // 32
    ngr = share // gr
    assert share % gr == 0
    mesh = plsc.VectorSubcoreMesh(core_axis_name="c", subcore_axis_name="s")

    def body(tab_hbm, idx_hbm, out_hbm, idx_v, rows, sem):
        w = lax.axis_index("s") * 2 + lax.axis_index("c")
        pltpu.sync_copy(idx_hbm.at[pl.ds(w * share, share)], idx_v)
        for g in range(ngr):
            pltpu.async_copy(tab_hbm.at[idx_v.at[pl.ds(g * gr, gr)]],
                             rows, sem).wait()
            pltpu.sync_copy(rows,
                            out_hbm.at[pl.ds(w * share + g * gr, gr)])

    def make(dty):
        return pl.kernel(
            body,
            out_type=jax.ShapeDtypeStruct((nrows, _D), dty),
            mesh=mesh,
            compiler_params=pltpu.CompilerParams(use_tc_tiling_on_sc=False, needs_layout_passes=False),
            scratch_types=[
                pltpu.VMEM((share,), jnp.int32),
                pltpu.VMEM((gr, _D), dty),
                pltpu.SemaphoreType.DMA,
            ],
        )
    return make(jnp.float32)


def _xg1_body(a_ref, s_ref, c_ref, o_ref):
    cnt = jnp.maximum(c_ref[:, :1], 1.0)
    o_ref[...] = a_ref[...] + s_ref[...] / cnt


def _rgcn_body(s_ref, c_ref, x_ref, coeff_ref, bases_ref, root_ref,
               bias_ref, o_ref, *, relu):
    mean = s_ref[...] / jnp.maximum(c_ref[...][..., :1], 1.0)
    out = jnp.dot(x_ref[...], root_ref[...],
                  preferred_element_type=jnp.float32)
    coeff = coeff_ref[...]
    for b in range(_NB):
        cb = coeff[:, b].reshape(_R, 1, 1)
        aggb = jnp.sum(mean * cb, axis=0)
        out = out + jnp.dot(aggb, bases_ref[b],
                            preferred_element_type=jnp.float32)
    out = out + bias_ref[...]
    if relu:
        out = jnp.maximum(out, 0.0)
    o_ref[...] = out


def _pred_body(xm_ref, se_ref, w_ref, o_ref):
    xm = xm_ref[...]
    se = jnp.clip(se_ref[...], 0.0, 1.0)
    w = jnp.clip(w_ref[...], 0.0, 1.0)
    xsq = (xm * xm) * w[:, 0][None, :]
    o_ref[...] = jax.nn.sigmoid(
        jnp.einsum('bsd,bd->bs', se, xsq,
                   preferred_element_type=jnp.float32))


def _pad_edges(src, key, e_pad, sentinel):
    e = src.shape[0]
    src_p = jnp.concatenate(
        [src, jnp.zeros((e_pad - e,), jnp.int32)])
    key_p = jnp.concatenate(
        [key, jnp.full((e_pad - e,), sentinel, jnp.int32)])
    packed = jnp.stack([src_p.reshape(-1, _CH), key_p.reshape(-1, _CH)],
                       axis=1)
    return packed.reshape(-1)


_E1_PAD = 13 * _NTILE * _CH
_E2_PAD = 10 * _NTILE * _CH
_NSLAB1 = 4
_NSLAB2 = 50
_KP1 = _NSLAB1 * _SLAB
_KP2 = _NSLAB2 * _SLAB

_seg1 = _make_segsum(_E1_PAD, _NSLAB1, 2, _NT)
_seg2a = _make_segsum(_E2_PAD, _NSLAB2, 25, _KP1)
_seg2b = _make_segsum(_E2_PAD, _NSLAB2, 25, _N2, with_cnt=False)
_gat_se = _make_gather(_B * _S, _GR)
_gat_xm = _make_gather(_B, _B // 32)


def kernel(all_node_embedding, bases1, coeff1, root1, bias1, bases2, coeff2,
           root2, bias2, weights, edge_index_g2, edge_type_g2, edge_index_g1,
           index_list, sample_index, sample_index_min):
    f32 = jnp.float32
    aemb = all_node_embedding
    ones_in = jnp.ones((_GR, 8), f32)
    zrows_in = jnp.zeros((_GR, _D), f32)
    zeros8_in = jnp.zeros((_GR, 8), f32)

    src1 = edge_index_g1[0].astype(jnp.int32)
    key1 = edge_index_g1[1].astype(jnp.int32)
    e1p = _pad_edges(src1, key1, _E1_PAD, _KP1)
    sum1, cnt1 = _seg1(aemb, e1p, ones_in, zrows_in, zeros8_in)

    aemb_p = jnp.concatenate(
        [aemb, jnp.zeros((_KP1 - _NT, _D), f32)])
    x_g1p = pl.pallas_call(
        _xg1_body,
        grid=(_KP1 // 800,),
        in_specs=[
            pl.BlockSpec((800, _D), lambda i: (i, 0)),
            pl.BlockSpec((800, _D), lambda i: (i, 0)),
            pl.BlockSpec((800, 8), lambda i: (i, 0)),
        ],
        out_specs=pl.BlockSpec((800, _D), lambda i: (i, 0)),
        out_shape=jax.ShapeDtypeStruct((_KP1, _D), f32),
    )(aemb_p, sum1, cnt1)

    src2 = edge_index_g2[0].astype(jnp.int32)
    key2 = (edge_type_g2.astype(jnp.int32) * _N2
            + edge_index_g2[1].astype(jnp.int32))
    e2p = _pad_edges(src2, key2, _E2_PAD, _KP2)

    bn = 400
    def rgcn_dense(sums, cnts, x_in, coeff, bases, root, bias, relu):
        body = functools.partial(_rgcn_body, relu=relu)
        return pl.pallas_call(
            body,
            grid=(_N2 // bn,),
            in_specs=[
                pl.BlockSpec((_R, bn, _D), lambda i: (0, i, 0)),
                pl.BlockSpec((_R, bn, 8), lambda i: (0, i, 0)),
                pl.BlockSpec((bn, _D), lambda i: (i, 0)),
                pl.BlockSpec((_R, _NB), lambda i: (0, 0)),
                pl.BlockSpec((_NB, _D, _D), lambda i: (0, 0, 0)),
                pl.BlockSpec((_D, _D), lambda i: (0, 0)),
                pl.BlockSpec((1, _D), lambda i: (0, 0)),
            ],
            out_specs=pl.BlockSpec((bn, _D), lambda i: (i, 0)),
            out_shape=jax.ShapeDtypeStruct((_N2, _D), f32),
        )(sums.reshape(_R, _N2, _D), cnts.reshape(_R, _N2, 8), x_in,
          coeff, bases, root, bias.reshape(1, _D))

    sum2, cnt2 = _seg2a(x_g1p, e2p, ones_in, zrows_in, zeros8_in)
    h = rgcn_dense(sum2, cnt2, x_g1p[:_N2], coeff1, bases1, root1, bias1,
                   True)
    sum3, _ = _seg2b(h, e2p, ones_in, zrows_in, zeros8_in)
    h = rgcn_dense(sum3, cnt2, h, coeff2, bases2, root2, bias2, False)

    se_idx = (sample_index_min.astype(jnp.int32) + _N2).reshape(-1)
    se_rows = _gat_se(x_g1p, se_idx)
    x_mini = _gat_xm(h, index_list.astype(jnp.int32))

    bb = 128
    out = pl.pallas_call(
        _pred_body,
        grid=(_B // bb,),
        in_specs=[
            pl.BlockSpec((bb, _D), lambda i: (i, 0)),
            pl.BlockSpec((bb, _S, _D), lambda i: (i, 0, 0)),
            pl.BlockSpec((_D, 1), lambda i: (0, 0)),
        ],
        out_specs=pl.BlockSpec((bb, _S), lambda i: (i, 0)),
        out_shape=jax.ShapeDtypeStruct((_B, _S), f32),
    )(x_mini, se_rows.reshape(_B, _S, _D), weights)
    return out
